# Initial kernel scaffold; baseline (speedup 1.0000x reference)
import jax
import jax.numpy as jnp
from jax.experimental import pallas as pl


def kernel(edge_index, in_feat, W0, b0, W1, b1, W2, b2, R0, rb0, R1, rb1, R2, rb2):
    raise NotImplementedError("write your pallas kernel here")


# speedup vs baseline: 4.8289x; 4.8289x over previous
import functools

import jax
import jax.numpy as jnp
from jax import lax
from jax.experimental import pallas as pl
from jax.experimental.pallas import tpu as pltpu
from jax.experimental.pallas import tpu_sc as plsc

NC = 2
NS = 16
NW = NC * NS
CH = 80


def _vector_mesh():
    return plsc.VectorSubcoreMesh(core_axis_name="c", subcore_axis_name="s")


@functools.partial(jax.jit, static_argnums=(2,))
def _sc_degrees(src, dst, n_pad):
    e = src.shape[0]
    span = e // NW
    n_ch = span // CH
    zb = n_pad // NS

    ones = jnp.ones((CH,), jnp.float32)
    zeros = jnp.zeros((zb,), jnp.float32)

    @functools.partial(
        pl.kernel,
        out_type=jax.ShapeDtypeStruct((NC, 2, n_pad), jnp.float32),
        mesh=_vector_mesh(),
        scratch_types=[
            pltpu.VMEM((CH,), jnp.int32),
            pltpu.VMEM((CH,), jnp.float32),
            pltpu.VMEM_SHARED((n_pad,), jnp.float32),
            pltpu.VMEM_SHARED((n_pad,), jnp.float32),
        ],
    )
    def k(src_hbm, dst_hbm, ones_hbm, zeros_hbm, out_hbm, iv, ones_v, od_acc, id_acc):
        cid = lax.axis_index("c")
        sid = lax.axis_index("s")
        wid = cid * NS + sid
        pltpu.sync_copy(ones_hbm, ones_v)
        pltpu.sync_copy(zeros_hbm, od_acc.at[pl.ds(sid * zb, zb)])
        pltpu.sync_copy(zeros_hbm, id_acc.at[pl.ds(sid * zb, zb)])
        plsc.subcore_barrier()
        base = wid * span

        @pl.loop(0, n_ch)
        def _(kk):
            off = base + kk * CH
            pltpu.sync_copy(src_hbm.at[pl.ds(off, CH)], iv)
            pltpu.sync_copy(ones_v, od_acc.at[iv], add=True)
            pltpu.sync_copy(dst_hbm.at[pl.ds(off, CH)], iv)
            pltpu.sync_copy(ones_v, id_acc.at[iv], add=True)

        plsc.subcore_barrier()
        sl = pl.ds(sid * zb, zb)
        pltpu.sync_copy(od_acc.at[sl], out_hbm.at[cid, 0, sl])
        pltpu.sync_copy(id_acc.at[sl], out_hbm.at[cid, 1, sl])

    return k(src, dst, ones, zeros)


@functools.partial(jax.jit, static_argnums=(3,))
def _sc_aggregate(h, src, dst, n_pad):
    d = h.shape[1]
    e = src.shape[0]
    span = e // NW
    n_ch = span // CH
    zb = n_pad // NS

    zeros = jnp.zeros((zb, d), jnp.float32)

    @functools.partial(
        pl.kernel,
        out_type=jax.ShapeDtypeStruct((NC, n_pad, d), jnp.float32),
        mesh=_vector_mesh(),
        scratch_types=[
            pltpu.VMEM((span,), jnp.int32),
            pltpu.VMEM((CH,), jnp.int32),
            pltpu.VMEM((CH, d), jnp.float32),
            pltpu.VMEM_SHARED((n_pad, d), jnp.float32),
        ],
        compiler_params=pltpu.CompilerParams(use_tc_tiling_on_sc=False),
    )
    def k(h_hbm, src_hbm, dst_hbm, zeros_hbm, out_hbm, srcv, dv, rows, acc):
        cid = lax.axis_index("c")
        sid = lax.axis_index("s")
        wid = cid * NS + sid
        pltpu.sync_copy(zeros_hbm, acc.at[pl.ds(sid * zb, zb)])
        base = wid * span
        pltpu.sync_copy(src_hbm.at[pl.ds(base, span)], srcv)
        plsc.subcore_barrier()

        @pl.loop(0, n_ch)
        def _(kk):
            pltpu.sync_copy(dst_hbm.at[pl.ds(base + kk * CH, CH)], dv)
            pltpu.sync_copy(h_hbm.at[srcv.at[pl.ds(kk * CH, CH)]], rows)
            pltpu.sync_copy(rows, acc.at[dv], add=True)

        plsc.subcore_barrier()
        sl = pl.ds(sid * zb, zb)
        pltpu.sync_copy(acc.at[sl], out_hbm.at[cid, sl])

    return k(h, src, dst, zeros)


RB = 2048


def _rsqrt_deg(dp_ref, which):
    deg = dp_ref[0, which, :] + dp_ref[1, which, :]
    return lax.rsqrt(jnp.maximum(deg, 1.0))[:, None]


def _tc_prep0(deg, x):
    n_pad = x.shape[0]
    grid = n_pad // RB

    def body(dp, x_ref, o_ref):
        o_ref[...] = x_ref[...] * _rsqrt_deg(dp, 0)

    return pl.pallas_call(
        body,
        grid=(grid,),
        in_specs=[
            pl.BlockSpec((2, 2, RB), lambda i: (0, 0, i)),
            pl.BlockSpec((RB, x.shape[1]), lambda i: (i, 0)),
        ],
        out_specs=pl.BlockSpec((RB, x.shape[1]), lambda i: (i, 0)),
        out_shape=jax.ShapeDtypeStruct(x.shape, jnp.float32),
    )(deg, x)


def _tc_mid(deg, aggp, w, b, w2=None):
    n_pad, din = aggp.shape[1], aggp.shape[2]
    dout = w.shape[1] if w2 is None else w2.shape[1]
    grid = n_pad // RB

    def body(dp, a_ref, w_ref, b_ref, *rest):
        o_ref = rest[-1]
        agg = (a_ref[0] + a_ref[1]) * _rsqrt_deg(dp, 1)
        t = jnp.tanh(
            lax.dot_general(agg, w_ref[...], (((1,), (0,)), ((), ())),
                            preferred_element_type=jnp.float32)
            + b_ref[...]
        )
        t = t * _rsqrt_deg(dp, 0)
        if w2 is not None:
            t = lax.dot_general(t, rest[0][...], (((1,), (0,)), ((), ())),
                                preferred_element_type=jnp.float32)
        o_ref[...] = t

    in_specs = [
        pl.BlockSpec((2, 2, RB), lambda i: (0, 0, i)),
        pl.BlockSpec((2, RB, din), lambda i: (0, i, 0)),
        pl.BlockSpec(w.shape, lambda i: (0, 0)),
        pl.BlockSpec((1, w.shape[1]), lambda i: (0, 0)),
    ]
    args = [deg, aggp, w, b.reshape(1, -1)]
    if w2 is not None:
        in_specs.append(pl.BlockSpec(w2.shape, lambda i: (0, 0)))
        args.append(w2)

    return pl.pallas_call(
        body,
        grid=(grid,),
        in_specs=in_specs,
        out_specs=pl.BlockSpec((RB, dout), lambda i: (i, 0)),
        out_shape=jax.ShapeDtypeStruct((n_pad, dout), jnp.float32),
    )(*args)


def _softmax(z):
    m = jnp.max(z, axis=1, keepdims=True)
    e = jnp.exp(z - m)
    return e / jnp.sum(e, axis=1, keepdims=True)


def _tc_final(deg, aggp, b2, r0, rb0, r1, rb1, r2, rb2):
    n_pad, d2 = aggp.shape[1], aggp.shape[2]
    grid = n_pad // RB

    def body(dp, a_ref, b2_ref, r0_ref, rb0_ref, r1_ref, rb1_ref, r2_ref,
             rb2_ref, h_ref, r_ref):
        z = (a_ref[0] + a_ref[1]) * _rsqrt_deg(dp, 1) + b2_ref[...]
        h = _softmax(z)
        h_ref[...] = h
        dot = lambda a, w: lax.dot_general(
            a, w, (((1,), (0,)), ((), ())), preferred_element_type=jnp.float32)
        r = jnp.tanh(dot(h, r0_ref[...]) + rb0_ref[...])
        r = jnp.tanh(dot(r, r1_ref[...]) + rb1_ref[...])
        r_ref[...] = _softmax(dot(r, r2_ref[...]) + rb2_ref[...])

    full = lambda a: pl.BlockSpec(a.shape, lambda i: (0, 0))
    row = lambda a: pl.BlockSpec((1, a.shape[0]), lambda i: (0, 0))
    return pl.pallas_call(
        body,
        grid=(grid,),
        in_specs=[
            pl.BlockSpec((2, 2, RB), lambda i: (0, 0, i)),
            pl.BlockSpec((2, RB, d2), lambda i: (0, i, 0)),
            row(b2), full(r0), row(rb0), full(r1), row(rb1), full(r2), row(rb2),
        ],
        out_specs=[
            pl.BlockSpec((RB, d2), lambda i: (i, 0)),
            pl.BlockSpec((RB, r2.shape[1]), lambda i: (i, 0)),
        ],
        out_shape=[
            jax.ShapeDtypeStruct((n_pad, d2), jnp.float32),
            jax.ShapeDtypeStruct((n_pad, r2.shape[1]), jnp.float32),
        ],
    )(deg, aggp, b2.reshape(1, -1), r0, rb0.reshape(1, -1), r1,
      rb1.reshape(1, -1), r2, rb2.reshape(1, -1))


def kernel(edge_index, in_feat, W0, b0, W1, b1, W2, b2, R0, rb0, R1, rb1, R2, rb2):
    n = in_feat.shape[0]
    n_pad = ((n + RB - 1) // RB) * RB
    src = edge_index[0]
    dst = edge_index[1]

    x = jnp.zeros((n_pad, in_feat.shape[1]), jnp.float32).at[:n].set(in_feat)

    deg = _sc_degrees(src, dst, n_pad)
    h0 = _tc_prep0(deg, x)
    agg0 = _sc_aggregate(h0, src, dst, n_pad)
    h1 = _tc_mid(deg, agg0, W0, b0)
    agg1 = _sc_aggregate(h1, src, dst, n_pad)
    h2 = _tc_mid(deg, agg1, W1, b1, w2=W2)
    agg2 = _sc_aggregate(h2, src, dst, n_pad)
    h, r = _tc_final(deg, agg2, b2, R0, rb0, R1, rb1, R2, rb2)
    return (h[:n], r[:n])
